# Initial kernel scaffold; baseline (speedup 1.0000x reference)
import jax
import jax.numpy as jnp
from jax.experimental import pallas as pl


def kernel(edge_idx, edge_features, node_features, W_e, b_e, W_n, b_n):
    raise NotImplementedError("write your pallas kernel here")


# speedup vs baseline: 3.4532x; 3.4532x over previous
import functools

import jax
import jax.numpy as jnp
from jax import lax
from jax.experimental import pallas as pl
from jax.experimental.pallas import tpu as pltpu
from jax.experimental.pallas import tpu_sc as plsc

N_NODES = 10000
N_EDGES = 320000
D_FEAT = 128
D_EDGE = 16

BLK = 128
NBLK = N_EDGES // BLK
NC = 2
NS = 16
NW = NC * NS
FULL_ROUNDS = NBLK // NW
REM = NBLK % NW
ROWS_PER_SUB = 640
N_ACC = NS * ROWS_PER_SUB


def _proj_pq_body(nf_ref, ws_ref, wr_ref, p_ref, q_ref):
    x = nf_ref[...]
    p_ref[...] = jnp.dot(x, ws_ref[...], preferred_element_type=jnp.float32)
    q_ref[...] = jnp.dot(x, wr_ref[...], preferred_element_type=jnp.float32)


def _proj_e_body(ef_ref, we_ref, be_ref, e_ref):
    e_ref[...] = (
        jnp.dot(ef_ref[...], we_ref[...], preferred_element_type=jnp.float32)
        + be_ref[...]
    )


def _sc_edge_body(sidx_hbm, ridx_hbm, p_hbm, q_hbm, e_hbm,
                  oe_hbm, sums_hbm, cnts_hbm,
                  sidx_v, ridx_v, pg, qg, eb, ob, onesb, zbuf,
                  sum_sh, cnt_sh):
    c = lax.axis_index("c")
    s = lax.axis_index("s")
    wid = c * NS + s

    @pl.loop(0, ROWS_PER_SUB)
    def _zero_fill(i):
        zbuf[i, :] = jnp.zeros((16,), jnp.float32)

    @pl.loop(0, BLK)
    def _ones_fill(i):
        onesb[i, :] = jnp.ones((16,), jnp.float32)

    pltpu.sync_copy(zbuf, sum_sh.at[pl.ds(s * ROWS_PER_SUB, ROWS_PER_SUB)])
    pltpu.sync_copy(zbuf, cnt_sh.at[pl.ds(s * ROWS_PER_SUB, ROWS_PER_SUB)])
    plsc.subcore_barrier()

    def process_block(b):
        e0 = b * BLK
        pltpu.sync_copy(sidx_hbm.at[b], sidx_v)
        pltpu.sync_copy(ridx_hbm.at[b], ridx_v)
        pltpu.sync_copy(p_hbm.at[sidx_v.at[0]], pg)
        pltpu.sync_copy(q_hbm.at[ridx_v.at[0]], qg)
        pltpu.sync_copy(e_hbm.at[pl.ds(e0, BLK)], eb)

        @pl.loop(0, BLK)
        def _compute(i):
            ob[i, :] = jnp.maximum(pg[i, :] + qg[i, :] + eb[i, :], 0.0)

        pltpu.sync_copy(ob, oe_hbm.at[pl.ds(e0, BLK)])
        pltpu.sync_copy(ob, sum_sh.at[ridx_v.at[0]], add=True)
        pltpu.sync_copy(onesb, cnt_sh.at[ridx_v.at[0]], add=True)

    @pl.loop(0, FULL_ROUNDS)
    def _main(t):
        process_block(t * NW + wid)

    @pl.when(wid < REM)
    def _tail():
        process_block(FULL_ROUNDS * NW + wid)

    plsc.subcore_barrier()

    src = pl.ds(s * ROWS_PER_SUB, ROWS_PER_SUB)
    dst = pl.ds(c * N_ACC + s * ROWS_PER_SUB, ROWS_PER_SUB)
    pltpu.sync_copy(sum_sh.at[src], sums_hbm.at[dst])
    pltpu.sync_copy(cnt_sh.at[src], cnts_hbm.at[dst])


def _node_body(nf_ref, sums_ref, cnts_ref, w1_ref, w2_ref, bn_ref, out_ref):
    ssum = sums_ref[0] + sums_ref[1]
    cnt = cnts_ref[0] + cnts_ref[1]
    mean = ssum / jnp.maximum(cnt, 1.0)
    y = (
        jnp.dot(nf_ref[...], w1_ref[...], preferred_element_type=jnp.float32)
        + jnp.dot(mean, w2_ref[...], preferred_element_type=jnp.float32)
        + bn_ref[...]
    )
    out_ref[...] = jnp.maximum(y, 0.0)


def kernel(edge_idx, edge_features, node_features, W_e, b_e, W_n, b_n):
    edge_idx = edge_idx.astype(jnp.int32)
    senders = edge_idx[:, 0].reshape(NBLK, 1, BLK)
    receivers = edge_idx[:, 1].reshape(NBLK, 1, BLK)

    W_s = W_e[:D_FEAT]
    W_r = W_e[D_FEAT:2 * D_FEAT]
    W_ed = W_e[2 * D_FEAT:]
    b_e2 = b_e.reshape(1, D_EDGE)
    W_n1 = W_n[:D_FEAT]
    W_n2 = W_n[D_FEAT:]
    b_n2 = b_n.reshape(1, D_FEAT)

    nb = 1000
    p, q = pl.pallas_call(
        _proj_pq_body,
        grid=(N_NODES // nb,),
        in_specs=[
            pl.BlockSpec((nb, D_FEAT), lambda i: (i, 0)),
            pl.BlockSpec((D_FEAT, D_EDGE), lambda i: (0, 0)),
            pl.BlockSpec((D_FEAT, D_EDGE), lambda i: (0, 0)),
        ],
        out_specs=[
            pl.BlockSpec((nb, D_EDGE), lambda i: (i, 0)),
            pl.BlockSpec((nb, D_EDGE), lambda i: (i, 0)),
        ],
        out_shape=[
            jax.ShapeDtypeStruct((N_NODES, D_EDGE), jnp.float32),
            jax.ShapeDtypeStruct((N_NODES, D_EDGE), jnp.float32),
        ],
    )(node_features, W_s, W_r)

    eb = 4000
    e_proj = pl.pallas_call(
        _proj_e_body,
        grid=(N_EDGES // eb,),
        in_specs=[
            pl.BlockSpec((eb, D_EDGE), lambda i: (i, 0)),
            pl.BlockSpec((D_EDGE, D_EDGE), lambda i: (0, 0)),
            pl.BlockSpec((1, D_EDGE), lambda i: (0, 0)),
        ],
        out_specs=pl.BlockSpec((eb, D_EDGE), lambda i: (i, 0)),
        out_shape=jax.ShapeDtypeStruct((N_EDGES, D_EDGE), jnp.float32),
    )(edge_features, W_ed, b_e2)

    mesh = plsc.VectorSubcoreMesh(core_axis_name="c", subcore_axis_name="s")
    sc_call = pl.kernel(
        _sc_edge_body,
        out_type=[
            jax.ShapeDtypeStruct((N_EDGES, D_EDGE), jnp.float32),
            jax.ShapeDtypeStruct((NC * N_ACC, D_EDGE), jnp.float32),
            jax.ShapeDtypeStruct((NC * N_ACC, D_EDGE), jnp.float32),
        ],
        mesh=mesh,
        compiler_params=pltpu.CompilerParams(use_tc_tiling_on_sc=False),
        scratch_types=[
            pltpu.VMEM((1, BLK), jnp.int32),
            pltpu.VMEM((1, BLK), jnp.int32),
            pltpu.VMEM((BLK, D_EDGE), jnp.float32),
            pltpu.VMEM((BLK, D_EDGE), jnp.float32),
            pltpu.VMEM((BLK, D_EDGE), jnp.float32),
            pltpu.VMEM((BLK, D_EDGE), jnp.float32),
            pltpu.VMEM((BLK, D_EDGE), jnp.float32),
            pltpu.VMEM((ROWS_PER_SUB, D_EDGE), jnp.float32),
            pltpu.VMEM_SHARED((N_ACC, D_EDGE), jnp.float32),
            pltpu.VMEM_SHARED((N_ACC, D_EDGE), jnp.float32),
        ],
    )
    new_edge, sums, cnts = sc_call(senders, receivers, p, q, e_proj)

    sums = sums.reshape(NC, N_ACC, D_EDGE)
    cnts = cnts.reshape(NC, N_ACC, D_EDGE)

    new_node = pl.pallas_call(
        _node_body,
        grid=(N_NODES // nb,),
        in_specs=[
            pl.BlockSpec((nb, D_FEAT), lambda i: (i, 0)),
            pl.BlockSpec((NC, nb, D_EDGE), lambda i: (0, i, 0)),
            pl.BlockSpec((NC, nb, D_EDGE), lambda i: (0, i, 0)),
            pl.BlockSpec((D_FEAT, D_FEAT), lambda i: (0, 0)),
            pl.BlockSpec((D_EDGE, D_FEAT), lambda i: (0, 0)),
            pl.BlockSpec((1, D_FEAT), lambda i: (0, 0)),
        ],
        out_specs=pl.BlockSpec((nb, D_FEAT), lambda i: (i, 0)),
        out_shape=jax.ShapeDtypeStruct((N_NODES, D_FEAT), jnp.float32),
    )(node_features, sums, cnts, W_n1, W_n2, b_n2)

    return new_edge, new_node


# speedup vs baseline: 4.3562x; 1.2615x over previous
import functools

import jax
import jax.numpy as jnp
from jax import lax
from jax.experimental import pallas as pl
from jax.experimental.pallas import tpu as pltpu
from jax.experimental.pallas import tpu_sc as plsc

N_NODES = 10000
N_EDGES = 320000
D_FEAT = 128
D_EDGE = 16

BLK = 128
NBLK = N_EDGES // BLK
PACK = 8
PACK_ROWS = BLK // PACK
NROWS_P = N_EDGES // PACK
NC = 2
NS = 16
NW = NC * NS
FULL_ROUNDS = NBLK // NW
REM = NBLK % NW
ROWS_PER_SUB = 640
N_ACC = NS * ROWS_PER_SUB


def _proj_pq_body(nf_ref, ws_ref, wr_ref, p_ref, q_ref):
    x = nf_ref[...]
    p_ref[...] = jnp.dot(x, ws_ref[...], preferred_element_type=jnp.float32)
    q_ref[...] = jnp.dot(x, wr_ref[...], preferred_element_type=jnp.float32)


def _proj_e_body(ef_ref, weblk_ref, be_ref, e_ref):
    e_ref[...] = (
        jnp.dot(ef_ref[...], weblk_ref[...],
                preferred_element_type=jnp.float32)
        + be_ref[...]
    )


def _sc_edge_body(sidx_hbm, ridx_hbm, p_hbm, q_hbm, e_hbm,
                  oe_hbm, sums_hbm, cnts_hbm,
                  sidx_v, ridx_v, pg, qg, eb, ob, ob2, onesb, zbuf,
                  sum_sh, cnt_sh):
    c = lax.axis_index("c")
    s = lax.axis_index("s")
    wid = c * NS + s

    @pl.loop(0, ROWS_PER_SUB)
    def _zero_fill(i):
        zbuf[i, :] = jnp.zeros((16,), jnp.float32)

    @pl.loop(0, BLK)
    def _ones_fill(i):
        onesb[i, :] = jnp.ones((16,), jnp.float32)

    pltpu.sync_copy(zbuf, sum_sh.at[pl.ds(s * ROWS_PER_SUB, ROWS_PER_SUB)])
    pltpu.sync_copy(zbuf, cnt_sh.at[pl.ds(s * ROWS_PER_SUB, ROWS_PER_SUB)])
    plsc.subcore_barrier()

    def process_block(b):
        r0 = b * PACK_ROWS
        pltpu.sync_copy(sidx_hbm.at[pl.ds(b, 1)], sidx_v)
        pltpu.sync_copy(ridx_hbm.at[pl.ds(b, 1)], ridx_v)
        pltpu.sync_copy(p_hbm.at[sidx_v.at[0]], pg)
        pltpu.sync_copy(q_hbm.at[ridx_v.at[0]], qg)
        pltpu.sync_copy(e_hbm.at[pl.ds(r0, PACK_ROWS)], eb)

        @pl.loop(0, PACK_ROWS)
        def _compute(rr):
            for jj in range(8):
                i = rr * 8 + jj
                sl = pl.ds(jj * D_EDGE, D_EDGE)
                v = jnp.maximum(pg[i, :] + qg[i, :] + eb[rr, sl], 0.0)
                ob[rr, sl] = v
                ob2[i, :] = v

        pltpu.sync_copy(ob, oe_hbm.at[pl.ds(r0, PACK_ROWS)])
        pltpu.sync_copy(ob2, sum_sh.at[ridx_v.at[0]], add=True)
        pltpu.sync_copy(onesb, cnt_sh.at[ridx_v.at[0]], add=True)

    @pl.loop(0, FULL_ROUNDS)
    def _main(t):
        process_block(t * NW + wid)

    @pl.when(wid < REM)
    def _tail():
        process_block(FULL_ROUNDS * NW + wid)

    plsc.subcore_barrier()

    src = pl.ds(s * ROWS_PER_SUB, ROWS_PER_SUB)
    dst = pl.ds(c * N_ACC + s * ROWS_PER_SUB, ROWS_PER_SUB)
    pltpu.sync_copy(sum_sh.at[src], sums_hbm.at[dst])
    pltpu.sync_copy(cnt_sh.at[src], cnts_hbm.at[dst])


def _node_body(nf_ref, sums_ref, cnts_ref, w1_ref, w2_ref, bn_ref, out_ref):
    ssum = sums_ref[0] + sums_ref[1]
    cnt = cnts_ref[0] + cnts_ref[1]
    mean = ssum / jnp.maximum(cnt, 1.0)
    y = (
        jnp.dot(nf_ref[...], w1_ref[...], preferred_element_type=jnp.float32)
        + jnp.dot(mean, w2_ref[...], preferred_element_type=jnp.float32)
        + bn_ref[...]
    )
    out_ref[...] = jnp.maximum(y, 0.0)


def kernel(edge_idx, edge_features, node_features, W_e, b_e, W_n, b_n):
    edge_idx = edge_idx.astype(jnp.int32)
    senders = edge_idx[:, 0].reshape(NBLK, BLK)
    receivers = edge_idx[:, 1].reshape(NBLK, BLK)

    W_s = W_e[:D_FEAT]
    W_r = W_e[D_FEAT:2 * D_FEAT]
    W_ed = W_e[2 * D_FEAT:]
    eye8 = jnp.eye(PACK, dtype=jnp.float32)
    W_blk = jnp.einsum("ab,ko->akbo", eye8, W_ed).reshape(PACK * D_EDGE,
                                                          PACK * D_EDGE)
    b_e_t = jnp.tile(b_e, PACK).reshape(1, PACK * D_EDGE)
    W_n1 = W_n[:D_FEAT]
    W_n2 = W_n[D_FEAT:]
    b_n2 = b_n.reshape(1, D_FEAT)

    nb = 1000
    p, q = pl.pallas_call(
        _proj_pq_body,
        grid=(N_NODES // nb,),
        in_specs=[
            pl.BlockSpec((nb, D_FEAT), lambda i: (i, 0)),
            pl.BlockSpec((D_FEAT, D_EDGE), lambda i: (0, 0)),
            pl.BlockSpec((D_FEAT, D_EDGE), lambda i: (0, 0)),
        ],
        out_specs=[
            pl.BlockSpec((nb, D_EDGE), lambda i: (i, 0)),
            pl.BlockSpec((nb, D_EDGE), lambda i: (i, 0)),
        ],
        out_shape=[
            jax.ShapeDtypeStruct((N_NODES, D_EDGE), jnp.float32),
            jax.ShapeDtypeStruct((N_NODES, D_EDGE), jnp.float32),
        ],
    )(node_features, W_s, W_r)

    ebn = 1000
    e_proj = pl.pallas_call(
        _proj_e_body,
        grid=(NROWS_P // ebn,),
        in_specs=[
            pl.BlockSpec((ebn, PACK * D_EDGE), lambda i: (i, 0)),
            pl.BlockSpec((PACK * D_EDGE, PACK * D_EDGE), lambda i: (0, 0)),
            pl.BlockSpec((1, PACK * D_EDGE), lambda i: (0, 0)),
        ],
        out_specs=pl.BlockSpec((ebn, PACK * D_EDGE), lambda i: (i, 0)),
        out_shape=jax.ShapeDtypeStruct((NROWS_P, PACK * D_EDGE), jnp.float32),
    )(edge_features.reshape(NROWS_P, PACK * D_EDGE), W_blk, b_e_t)

    mesh = plsc.VectorSubcoreMesh(core_axis_name="c", subcore_axis_name="s")
    sc_call = pl.kernel(
        _sc_edge_body,
        out_type=[
            jax.ShapeDtypeStruct((NROWS_P, PACK * D_EDGE), jnp.float32),
            jax.ShapeDtypeStruct((NC * N_ACC, D_EDGE), jnp.float32),
            jax.ShapeDtypeStruct((NC * N_ACC, D_EDGE), jnp.float32),
        ],
        mesh=mesh,
        compiler_params=pltpu.CompilerParams(use_tc_tiling_on_sc=False),
        scratch_types=[
            pltpu.VMEM((1, BLK), jnp.int32),
            pltpu.VMEM((1, BLK), jnp.int32),
            pltpu.VMEM((BLK, D_EDGE), jnp.float32),
            pltpu.VMEM((BLK, D_EDGE), jnp.float32),
            pltpu.VMEM((PACK_ROWS, PACK * D_EDGE), jnp.float32),
            pltpu.VMEM((PACK_ROWS, PACK * D_EDGE), jnp.float32),
            pltpu.VMEM((BLK, D_EDGE), jnp.float32),
            pltpu.VMEM((BLK, D_EDGE), jnp.float32),
            pltpu.VMEM((ROWS_PER_SUB, D_EDGE), jnp.float32),
            pltpu.VMEM_SHARED((N_ACC, D_EDGE), jnp.float32),
            pltpu.VMEM_SHARED((N_ACC, D_EDGE), jnp.float32),
        ],
    )
    new_edge_p, sums, cnts = sc_call(senders, receivers, p, q, e_proj)

    new_edge = new_edge_p.reshape(N_EDGES, D_EDGE)
    sums = sums.reshape(NC, N_ACC, D_EDGE)
    cnts = cnts.reshape(NC, N_ACC, D_EDGE)

    new_node = pl.pallas_call(
        _node_body,
        grid=(N_NODES // nb,),
        in_specs=[
            pl.BlockSpec((nb, D_FEAT), lambda i: (i, 0)),
            pl.BlockSpec((NC, nb, D_EDGE), lambda i: (0, i, 0)),
            pl.BlockSpec((NC, nb, D_EDGE), lambda i: (0, i, 0)),
            pl.BlockSpec((D_FEAT, D_FEAT), lambda i: (0, 0)),
            pl.BlockSpec((D_EDGE, D_FEAT), lambda i: (0, 0)),
            pl.BlockSpec((1, D_FEAT), lambda i: (0, 0)),
        ],
        out_specs=pl.BlockSpec((nb, D_FEAT), lambda i: (i, 0)),
        out_shape=jax.ShapeDtypeStruct((N_NODES, D_FEAT), jnp.float32),
    )(node_features, sums, cnts, W_n1, W_n2, b_n2)

    return new_edge, new_node


# speedup vs baseline: 5.1686x; 1.1865x over previous
import functools

import jax
import jax.numpy as jnp
from jax import lax
from jax.experimental import pallas as pl
from jax.experimental.pallas import tpu as pltpu
from jax.experimental.pallas import tpu_sc as plsc

N_NODES = 10000
N_EDGES = 320000
D_FEAT = 128
D_EDGE = 16

BLK = 128
NBLK = N_EDGES // BLK
PACK = 8
PACK_ROWS = BLK // PACK
NROWS_P = N_EDGES // PACK
NC = 2
NS = 16
NW = NC * NS
FULL_ROUNDS = NBLK // NW
REM = NBLK % NW
SUPER = 6
ROWS_PER_SUB = 640
N_ACC = NS * ROWS_PER_SUB


def _proj_pq_body(nf_ref, ws_ref, wr_ref, p_ref, q_ref):
    x = nf_ref[...]
    p_ref[...] = jnp.dot(x, ws_ref[...], preferred_element_type=jnp.float32)
    q_ref[...] = jnp.dot(x, wr_ref[...], preferred_element_type=jnp.float32)


def _proj_e_body(ef_ref, weblk_ref, be_ref, e_ref):
    e_ref[...] = (
        jnp.dot(ef_ref[...], weblk_ref[...],
                preferred_element_type=jnp.float32)
        + be_ref[...]
    )


def _sc_edge_body(sidx_hbm, ridx_hbm, p_hbm, q_hbm, e_hbm,
                  oe_hbm, sums_hbm, cnts_hbm, *scr):
    sidx_v = scr[0:6]
    ridx_v = scr[6:12]
    pg_v = scr[12:18]
    qg_v = scr[18:24]
    eb_v = scr[24:30]
    ob_v = scr[30:36]
    ob2_v = scr[36:42]
    onesb, zbuf, sum_sh, cnt_sh = scr[42:46]
    sem_idx = scr[46:52]
    sem_in = scr[52:58]
    sem_out = scr[58:64]

    c = lax.axis_index("c")
    s = lax.axis_index("s")
    wid = c * NS + s

    @pl.loop(0, ROWS_PER_SUB)
    def _zero_fill(i):
        zbuf[i, :] = jnp.zeros((16,), jnp.float32)

    @pl.loop(0, BLK)
    def _ones_fill(i):
        onesb[i, :] = jnp.ones((16,), jnp.float32)

    pltpu.sync_copy(zbuf, sum_sh.at[pl.ds(s * ROWS_PER_SUB, ROWS_PER_SUB)])
    pltpu.sync_copy(zbuf, cnt_sh.at[pl.ds(s * ROWS_PER_SUB, ROWS_PER_SUB)])
    plsc.subcore_barrier()

    def compute(k):
        pg, qg, ebf, ob, ob2 = pg_v[k], qg_v[k], eb_v[k], ob_v[k], ob2_v[k]

        @pl.loop(0, PACK_ROWS)
        def _compute(rr):
            for jj in range(PACK):
                i = rr * PACK + jj
                sl = pl.ds(jj * D_EDGE, D_EDGE)
                v = jnp.maximum(
                    pg[i, :] + qg[i, :] + ebf[rr, sl], 0.0)
                ob[rr, sl] = v
                ob2[i, :] = v

    def blk(t):
        return t * NW + wid

    @pl.loop(0, FULL_ROUNDS)
    def _main(t):
        b = blk(t)
        pltpu.sync_copy(sidx_hbm.at[pl.ds(b, 1)], sidx_v[0])
        pltpu.sync_copy(ridx_hbm.at[pl.ds(b, 1)], ridx_v[0])
        d1 = pltpu.async_copy(p_hbm.at[sidx_v[0].at[0]], pg_v[0], sem_in[0])
        d2 = pltpu.async_copy(q_hbm.at[ridx_v[0].at[0]], qg_v[0], sem_in[1])
        d3 = pltpu.async_copy(e_hbm.at[pl.ds(b * PACK_ROWS, PACK_ROWS)],
                              eb_v[0], sem_in[2])
        d1.wait()
        d2.wait()
        d3.wait()
        compute(0)
        pltpu.sync_copy(ob_v[0], oe_hbm.at[pl.ds(b * PACK_ROWS, PACK_ROWS)])
        pltpu.sync_copy(ob2_v[0], sum_sh.at[ridx_v[0].at[0]], add=True)
        pltpu.sync_copy(onesb, cnt_sh.at[ridx_v[0].at[0]], add=True)

    @pl.when(wid < REM)
    def _tail():
        b = FULL_ROUNDS * NW + wid
        r0 = b * PACK_ROWS
        pltpu.sync_copy(sidx_hbm.at[pl.ds(b, 1)], sidx_v[0])
        pltpu.sync_copy(ridx_hbm.at[pl.ds(b, 1)], ridx_v[0])
        pltpu.sync_copy(p_hbm.at[sidx_v[0].at[0]], pg_v[0])
        pltpu.sync_copy(q_hbm.at[ridx_v[0].at[0]], qg_v[0])
        pltpu.sync_copy(e_hbm.at[pl.ds(r0, PACK_ROWS)], eb_v[0])
        compute(0)
        pltpu.sync_copy(ob_v[0], oe_hbm.at[pl.ds(r0, PACK_ROWS)])
        pltpu.sync_copy(ob2_v[0], sum_sh.at[ridx_v[0].at[0]], add=True)
        pltpu.sync_copy(onesb, cnt_sh.at[ridx_v[0].at[0]], add=True)

    plsc.subcore_barrier()

    src = pl.ds(s * ROWS_PER_SUB, ROWS_PER_SUB)
    dst = pl.ds(c * N_ACC + s * ROWS_PER_SUB, ROWS_PER_SUB)
    pltpu.sync_copy(sum_sh.at[src], sums_hbm.at[dst])
    pltpu.sync_copy(cnt_sh.at[src], cnts_hbm.at[dst])


def _node_body(nf_ref, sums_ref, cnts_ref, w1_ref, w2_ref, bn_ref, out_ref):
    ssum = sums_ref[0] + sums_ref[1]
    cnt = cnts_ref[0] + cnts_ref[1]
    mean = ssum / jnp.maximum(cnt, 1.0)
    y = (
        jnp.dot(nf_ref[...], w1_ref[...], preferred_element_type=jnp.float32)
        + jnp.dot(mean, w2_ref[...], preferred_element_type=jnp.float32)
        + bn_ref[...]
    )
    out_ref[...] = jnp.maximum(y, 0.0)


def kernel(edge_idx, edge_features, node_features, W_e, b_e, W_n, b_n):
    edge_idx = edge_idx.astype(jnp.int32)
    senders = edge_idx[:, 0].reshape(NBLK, BLK)
    receivers = edge_idx[:, 1].reshape(NBLK, BLK)

    W_s = W_e[:D_FEAT]
    W_r = W_e[D_FEAT:2 * D_FEAT]
    W_ed = W_e[2 * D_FEAT:]
    eye8 = jnp.eye(PACK, dtype=jnp.float32)
    W_blk = jnp.einsum("ab,ko->akbo", eye8, W_ed).reshape(PACK * D_EDGE,
                                                          PACK * D_EDGE)
    b_e_t = jnp.tile(b_e, PACK).reshape(1, PACK * D_EDGE)
    W_n1 = W_n[:D_FEAT]
    W_n2 = W_n[D_FEAT:]
    b_n2 = b_n.reshape(1, D_FEAT)

    nb = 1000
    p, q = pl.pallas_call(
        _proj_pq_body,
        grid=(N_NODES // nb,),
        in_specs=[
            pl.BlockSpec((nb, D_FEAT), lambda i: (i, 0)),
            pl.BlockSpec((D_FEAT, D_EDGE), lambda i: (0, 0)),
            pl.BlockSpec((D_FEAT, D_EDGE), lambda i: (0, 0)),
        ],
        out_specs=[
            pl.BlockSpec((nb, D_EDGE), lambda i: (i, 0)),
            pl.BlockSpec((nb, D_EDGE), lambda i: (i, 0)),
        ],
        out_shape=[
            jax.ShapeDtypeStruct((N_NODES, D_EDGE), jnp.float32),
            jax.ShapeDtypeStruct((N_NODES, D_EDGE), jnp.float32),
        ],
    )(node_features, W_s, W_r)

    ebn = 1000
    e_proj = pl.pallas_call(
        _proj_e_body,
        grid=(NROWS_P // ebn,),
        in_specs=[
            pl.BlockSpec((ebn, PACK * D_EDGE), lambda i: (i, 0)),
            pl.BlockSpec((PACK * D_EDGE, PACK * D_EDGE), lambda i: (0, 0)),
            pl.BlockSpec((1, PACK * D_EDGE), lambda i: (0, 0)),
        ],
        out_specs=pl.BlockSpec((ebn, PACK * D_EDGE), lambda i: (i, 0)),
        out_shape=jax.ShapeDtypeStruct((NROWS_P, PACK * D_EDGE), jnp.float32),
    )(edge_features.reshape(NROWS_P, PACK * D_EDGE), W_blk, b_e_t)

    mesh = plsc.VectorSubcoreMesh(core_axis_name="c", subcore_axis_name="s")
    sc_call = pl.kernel(
        _sc_edge_body,
        out_type=[
            jax.ShapeDtypeStruct((NROWS_P, PACK * D_EDGE), jnp.float32),
            jax.ShapeDtypeStruct((NC * N_ACC, D_EDGE), jnp.float32),
            jax.ShapeDtypeStruct((NC * N_ACC, D_EDGE), jnp.float32),
        ],
        mesh=mesh,
        compiler_params=pltpu.CompilerParams(use_tc_tiling_on_sc=False),
        scratch_types=(
            [pltpu.VMEM((1, BLK), jnp.int32)] * 12
            + [pltpu.VMEM((BLK, D_EDGE), jnp.float32)] * 12
            + [pltpu.VMEM((PACK_ROWS, PACK * D_EDGE), jnp.float32)] * 12
            + [pltpu.VMEM((BLK, D_EDGE), jnp.float32)] * 6
            + [
                pltpu.VMEM((BLK, D_EDGE), jnp.float32),
                pltpu.VMEM((ROWS_PER_SUB, D_EDGE), jnp.float32),
                pltpu.VMEM_SHARED((N_ACC, D_EDGE), jnp.float32),
                pltpu.VMEM_SHARED((N_ACC, D_EDGE), jnp.float32),
            ]
            + [pltpu.SemaphoreType.DMA] * 18
        ),
    )
    new_edge_p, sums, cnts = sc_call(senders, receivers, p, q, e_proj)

    new_edge = new_edge_p.reshape(N_EDGES, D_EDGE)
    sums = sums.reshape(NC, N_ACC, D_EDGE)
    cnts = cnts.reshape(NC, N_ACC, D_EDGE)

    new_node = pl.pallas_call(
        _node_body,
        grid=(N_NODES // nb,),
        in_specs=[
            pl.BlockSpec((nb, D_FEAT), lambda i: (i, 0)),
            pl.BlockSpec((NC, nb, D_EDGE), lambda i: (0, i, 0)),
            pl.BlockSpec((NC, nb, D_EDGE), lambda i: (0, i, 0)),
            pl.BlockSpec((D_FEAT, D_FEAT), lambda i: (0, 0)),
            pl.BlockSpec((D_EDGE, D_FEAT), lambda i: (0, 0)),
            pl.BlockSpec((1, D_FEAT), lambda i: (0, 0)),
        ],
        out_specs=pl.BlockSpec((nb, D_FEAT), lambda i: (i, 0)),
        out_shape=jax.ShapeDtypeStruct((N_NODES, D_FEAT), jnp.float32),
    )(node_features, sums, cnts, W_n1, W_n2, b_n2)

    return new_edge, new_node


# speedup vs baseline: 5.9262x; 1.1466x over previous
import functools

import jax
import jax.numpy as jnp
from jax import lax
from jax.experimental import pallas as pl
from jax.experimental.pallas import tpu as pltpu
from jax.experimental.pallas import tpu_sc as plsc

N_NODES = 10000
N_EDGES = 320000
D_FEAT = 128
D_EDGE = 16

BLK = 128
NBLK = N_EDGES // BLK
PACK = 8
PACK_ROWS = BLK // PACK
NROWS_P = N_EDGES // PACK
NC = 2
NS = 16
NW = NC * NS
FULL_ROUNDS = NBLK // NW
REM = NBLK % NW
SUPER = 6
ROWS_PER_SUB = 640
N_ACC = NS * ROWS_PER_SUB


def _proj_pq_body(nf_ref, ws_ref, wr_ref, p_ref, q_ref):
    x = nf_ref[...]
    p_ref[...] = jnp.dot(x, ws_ref[...], preferred_element_type=jnp.float32)
    q_ref[...] = jnp.dot(x, wr_ref[...], preferred_element_type=jnp.float32)


def _proj_e_body(ef_ref, weblk_ref, be_ref, e_ref):
    e_ref[...] = (
        jnp.dot(ef_ref[...], weblk_ref[...],
                preferred_element_type=jnp.float32)
        + be_ref[...]
    )


def _sc_edge_body(sidx_hbm, ridx_hbm, p_hbm, q_hbm, e_hbm,
                  oe_hbm, acc_hbm, *scr):
    sidx_v = scr[0:6]
    ridx_v = scr[6:12]
    pg_v = scr[12:18]
    qg_v = scr[18:24]
    eb_v = scr[24:30]
    ob_v = scr[30:36]
    ob2_v = scr[36:42]
    zbuf, acc_sh = scr[42:44]
    sem_idx = scr[44:50]
    sem_in = scr[50:56]
    sem_out = scr[56:62]

    c = lax.axis_index("c")
    s = lax.axis_index("s")
    wid = c * NS + s

    zero16 = jnp.zeros((16,), jnp.float32)
    one16 = jnp.ones((16,), jnp.float32)
    lo = pl.ds(0, D_EDGE)
    hi = pl.ds(D_EDGE, D_EDGE)

    @pl.loop(0, ROWS_PER_SUB)
    def _zero_fill(i):
        zbuf[i, lo] = zero16
        zbuf[i, hi] = zero16

    @pl.loop(0, BLK)
    def _ones_fill(i):
        for k in range(SUPER):
            ob2_v[k][i, hi] = one16

    pltpu.sync_copy(zbuf, acc_sh.at[pl.ds(s * ROWS_PER_SUB, ROWS_PER_SUB)])
    plsc.subcore_barrier()

    def compute(k):
        pg, qg, ebf, ob, ob2 = pg_v[k], qg_v[k], eb_v[k], ob_v[k], ob2_v[k]

        @pl.loop(0, PACK_ROWS)
        def _compute(rr):
            for jj in range(PACK):
                i = rr * PACK + jj
                sl = pl.ds(jj * D_EDGE, D_EDGE)
                v = jnp.maximum(
                    pg[i, :] + qg[i, :] + ebf[rr, sl], 0.0)
                ob[rr, sl] = v
                ob2[i, lo] = v

    def blk(t):
        return t * NW + wid

    @pl.loop(0, FULL_ROUNDS // SUPER)
    def _main(g):
        t0 = g * SUPER
        d_idx = []
        for k in range(SUPER):
            b = blk(t0 + k)
            d_idx.append((
                pltpu.async_copy(sidx_hbm.at[pl.ds(b, 1)], sidx_v[k],
                                 sem_idx[k]),
                pltpu.async_copy(ridx_hbm.at[pl.ds(b, 1)], ridx_v[k],
                                 sem_idx[k]),
            ))
        d_in = []
        for k in range(SUPER):
            b = blk(t0 + k)
            for d in d_idx[k]:
                d.wait()
            d_in.append((
                pltpu.async_copy(p_hbm.at[sidx_v[k].at[0]], pg_v[k],
                                 sem_in[k]),
                pltpu.async_copy(q_hbm.at[ridx_v[k].at[0]], qg_v[k],
                                 sem_in[k]),
                pltpu.async_copy(e_hbm.at[pl.ds(b * PACK_ROWS, PACK_ROWS)],
                                 eb_v[k], sem_in[k]),
            ))
        d_out = []
        for k in range(SUPER):
            b = blk(t0 + k)
            for d in d_in[k]:
                d.wait()
            compute(k)
            d_out.append(
                pltpu.async_copy(ob_v[k],
                                 oe_hbm.at[pl.ds(b * PACK_ROWS, PACK_ROWS)],
                                 sem_out[k]))
            pltpu.sync_copy(ob2_v[k], acc_sh.at[ridx_v[k].at[0]], add=True)
        for d in d_out:
            d.wait()

    @pl.when(wid < REM)
    def _tail():
        b = FULL_ROUNDS * NW + wid
        r0 = b * PACK_ROWS
        pltpu.sync_copy(sidx_hbm.at[pl.ds(b, 1)], sidx_v[0])
        pltpu.sync_copy(ridx_hbm.at[pl.ds(b, 1)], ridx_v[0])
        pltpu.sync_copy(p_hbm.at[sidx_v[0].at[0]], pg_v[0])
        pltpu.sync_copy(q_hbm.at[ridx_v[0].at[0]], qg_v[0])
        pltpu.sync_copy(e_hbm.at[pl.ds(r0, PACK_ROWS)], eb_v[0])
        compute(0)
        pltpu.sync_copy(ob_v[0], oe_hbm.at[pl.ds(r0, PACK_ROWS)])
        pltpu.sync_copy(ob2_v[0], acc_sh.at[ridx_v[0].at[0]], add=True)

    plsc.subcore_barrier()

    src = pl.ds(s * ROWS_PER_SUB, ROWS_PER_SUB)
    dst = pl.ds(c * N_ACC + s * ROWS_PER_SUB, ROWS_PER_SUB)
    pltpu.sync_copy(acc_sh.at[src], acc_hbm.at[dst])


def _node_body(nf_ref, acc_ref, w1_ref, w2_ref, bn_ref, out_ref):
    ssum = acc_ref[0, :, :D_EDGE] + acc_ref[1, :, :D_EDGE]
    cnt = acc_ref[0, :, D_EDGE:] + acc_ref[1, :, D_EDGE:]
    mean = ssum / jnp.maximum(cnt, 1.0)
    y = (
        jnp.dot(nf_ref[...], w1_ref[...], preferred_element_type=jnp.float32)
        + jnp.dot(mean, w2_ref[...], preferred_element_type=jnp.float32)
        + bn_ref[...]
    )
    out_ref[...] = jnp.maximum(y, 0.0)


def kernel(edge_idx, edge_features, node_features, W_e, b_e, W_n, b_n):
    edge_idx = edge_idx.astype(jnp.int32)
    senders = edge_idx[:, 0].reshape(NBLK, BLK)
    receivers = edge_idx[:, 1].reshape(NBLK, BLK)

    W_s = W_e[:D_FEAT]
    W_r = W_e[D_FEAT:2 * D_FEAT]
    W_ed = W_e[2 * D_FEAT:]
    eye8 = jnp.eye(PACK, dtype=jnp.float32)
    W_blk = jnp.einsum("ab,ko->akbo", eye8, W_ed).reshape(PACK * D_EDGE,
                                                          PACK * D_EDGE)
    b_e_t = jnp.tile(b_e, PACK).reshape(1, PACK * D_EDGE)
    W_n1 = W_n[:D_FEAT]
    W_n2 = W_n[D_FEAT:]
    b_n2 = b_n.reshape(1, D_FEAT)

    nb = 1000
    p, q = pl.pallas_call(
        _proj_pq_body,
        grid=(N_NODES // nb,),
        in_specs=[
            pl.BlockSpec((nb, D_FEAT), lambda i: (i, 0)),
            pl.BlockSpec((D_FEAT, D_EDGE), lambda i: (0, 0)),
            pl.BlockSpec((D_FEAT, D_EDGE), lambda i: (0, 0)),
        ],
        out_specs=[
            pl.BlockSpec((nb, D_EDGE), lambda i: (i, 0)),
            pl.BlockSpec((nb, D_EDGE), lambda i: (i, 0)),
        ],
        out_shape=[
            jax.ShapeDtypeStruct((N_NODES, D_EDGE), jnp.float32),
            jax.ShapeDtypeStruct((N_NODES, D_EDGE), jnp.float32),
        ],
    )(node_features, W_s, W_r)

    ebn = 1000
    e_proj = pl.pallas_call(
        _proj_e_body,
        grid=(NROWS_P // ebn,),
        in_specs=[
            pl.BlockSpec((ebn, PACK * D_EDGE), lambda i: (i, 0)),
            pl.BlockSpec((PACK * D_EDGE, PACK * D_EDGE), lambda i: (0, 0)),
            pl.BlockSpec((1, PACK * D_EDGE), lambda i: (0, 0)),
        ],
        out_specs=pl.BlockSpec((ebn, PACK * D_EDGE), lambda i: (i, 0)),
        out_shape=jax.ShapeDtypeStruct((NROWS_P, PACK * D_EDGE), jnp.float32),
    )(edge_features.reshape(NROWS_P, PACK * D_EDGE), W_blk, b_e_t)

    mesh = plsc.VectorSubcoreMesh(core_axis_name="c", subcore_axis_name="s")
    sc_call = pl.kernel(
        _sc_edge_body,
        out_type=[
            jax.ShapeDtypeStruct((NROWS_P, PACK * D_EDGE), jnp.float32),
            jax.ShapeDtypeStruct((NC * N_ACC, 2 * D_EDGE), jnp.float32),
        ],
        mesh=mesh,
        compiler_params=pltpu.CompilerParams(use_tc_tiling_on_sc=False),
        scratch_types=(
            [pltpu.VMEM((1, BLK), jnp.int32)] * 12
            + [pltpu.VMEM((BLK, D_EDGE), jnp.float32)] * 12
            + [pltpu.VMEM((PACK_ROWS, PACK * D_EDGE), jnp.float32)] * 12
            + [pltpu.VMEM((BLK, 2 * D_EDGE), jnp.float32)] * 6
            + [
                pltpu.VMEM((ROWS_PER_SUB, 2 * D_EDGE), jnp.float32),
                pltpu.VMEM_SHARED((N_ACC, 2 * D_EDGE), jnp.float32),
            ]
            + [pltpu.SemaphoreType.DMA] * 18
        ),
    )
    new_edge_p, acc = sc_call(senders, receivers, p, q, e_proj)

    new_edge = new_edge_p.reshape(N_EDGES, D_EDGE)
    acc = acc.reshape(NC, N_ACC, 2 * D_EDGE)

    new_node = pl.pallas_call(
        _node_body,
        grid=(N_NODES // nb,),
        in_specs=[
            pl.BlockSpec((nb, D_FEAT), lambda i: (i, 0)),
            pl.BlockSpec((NC, nb, 2 * D_EDGE), lambda i: (0, i, 0)),
            pl.BlockSpec((D_FEAT, D_FEAT), lambda i: (0, 0)),
            pl.BlockSpec((D_EDGE, D_FEAT), lambda i: (0, 0)),
            pl.BlockSpec((1, D_FEAT), lambda i: (0, 0)),
        ],
        out_specs=pl.BlockSpec((nb, D_FEAT), lambda i: (i, 0)),
        out_shape=jax.ShapeDtypeStruct((N_NODES, D_FEAT), jnp.float32),
    )(node_features, acc, W_n1, W_n2, b_n2)

    return new_edge, new_node
